# baseline (speedup 1.0000x reference)
import functools

import jax
import jax.numpy as jnp
from jax import lax
from jax.experimental import pallas as pl
from jax.experimental.pallas import tpu as pltpu
from jax.experimental.pallas import tpu_sc as plsc

D = 64
L = 16
NW = 32
CHUNK = 128


def _dot_body(uidx_hbm, iidx_hbm, ufac_hbm, ifac_hbm, out_hbm,
              uidx_v, iidx_v, urows_v, irows_v, out_v, sem):
    wid = lax.axis_index("s") * 2 + lax.axis_index("c")
    bpw = out_v.shape[0]
    nchunks = bpw // CHUNK
    base = wid * bpw

    pltpu.sync_copy(uidx_hbm.at[wid], uidx_v)
    pltpu.sync_copy(iidx_hbm.at[wid], iidx_v)

    copies = []
    for j in range(nchunks):
        copies.append(pltpu.async_copy(
            ufac_hbm.at[uidx_v.at[j]],
            urows_v.at[pl.ds(j * CHUNK, CHUNK)], sem))
        copies.append(pltpu.async_copy(
            ifac_hbm.at[iidx_v.at[j]],
            irows_v.at[pl.ds(j * CHUNK, CHUNK)], sem))
    for cp in copies:
        cp.wait()

    zero = jnp.zeros((L,), jnp.float32)
    for blk in range(bpw // L):
        rvec = blk * L + lax.iota(jnp.int32, L)

        def d_step(i, accs, rvec=rvec):
            new = []
            for k in range(4):
                dcol = jnp.full((L,), i * 4 + k, jnp.int32)
                uu = plsc.load_gather(urows_v, [rvec, dcol])
                vv = plsc.load_gather(irows_v, [rvec, dcol])
                new.append(accs[k] + uu * vv)
            return tuple(new)

        a0, a1, a2, a3 = lax.fori_loop(
            0, D // 4, d_step, (zero, zero, zero, zero), unroll=8)
        out_v[pl.ds(blk * L, L)] = (a0 + a1) + (a2 + a3)

    pltpu.sync_copy(out_v, out_hbm.at[pl.ds(base, bpw)])


def kernel(user_item_tuple, user_factors, item_factors):
    batch = user_item_tuple.shape[0]
    bpw = batch // NW
    nchunks = bpw // CHUNK

    uit = user_item_tuple.astype(jnp.int32)
    u_idx = uit[:, 0].reshape(NW, nchunks, CHUNK)
    i_idx = uit[:, 1].reshape(NW, nchunks, CHUNK)

    mesh = plsc.VectorSubcoreMesh(core_axis_name="c", subcore_axis_name="s")
    run = functools.partial(
        pl.kernel,
        out_type=jax.ShapeDtypeStruct((batch,), jnp.float32),
        mesh=mesh,
        compiler_params=pltpu.CompilerParams(
            needs_layout_passes=False, use_tc_tiling_on_sc=False),
        scratch_types=[
            pltpu.VMEM((nchunks, CHUNK), jnp.int32),
            pltpu.VMEM((nchunks, CHUNK), jnp.int32),
            pltpu.VMEM((bpw, D), jnp.float32),
            pltpu.VMEM((bpw, D), jnp.float32),
            pltpu.VMEM((bpw,), jnp.float32),
            pltpu.SemaphoreType.DMA,
        ],
    )(_dot_body)
    return run(u_idx, i_idx, user_factors, item_factors)


# speedup vs baseline: 2.4281x; 2.4281x over previous
import functools

import jax
import jax.numpy as jnp
from jax import lax
from jax.experimental import pallas as pl
from jax.experimental.pallas import tpu as pltpu
from jax.experimental.pallas import tpu_sc as plsc

D = 64
L = 16
NW = 32
C = 32


def _take(v, idx):
    return jnp.take_along_axis(v, idx, axis=0, mode="promise_in_bounds")


def _dot_body(tu_hbm, su_hbm, ti_hbm, si_hbm, ufac_hbm, ifac_hbm, out_hbm,
              tu_v, su_v, ti_v, si_v, ut_v, it_v, out_v, sem):
    wid = lax.axis_index("s") * 2 + lax.axis_index("c")
    bpw = out_v.shape[0]
    nchunks = bpw // C
    base = wid * bpw

    pltpu.sync_copy(tu_hbm.at[pl.ds(base, bpw)], tu_v)
    pltpu.sync_copy(su_hbm.at[pl.ds(base, bpw)], su_v)
    pltpu.sync_copy(ti_hbm.at[pl.ds(base, bpw)], ti_v)
    pltpu.sync_copy(si_hbm.at[pl.ds(base, bpw)], si_v)

    lane = lax.iota(jnp.int32, L)
    xor_idx = [lane ^ sh for sh in (8, 4, 2, 1)]
    zero = jnp.zeros((L,), jnp.float32)

    def chunk_body(k, carry):
        copies = []
        for blk in range(C // L):
            off = k * C + blk * L
            tub = tu_v[pl.ds(off, L)]
            sub = su_v[pl.ds(off, L)]
            tib = ti_v[pl.ds(off, L)]
            sib = si_v[pl.ds(off, L)]
            for j in range(L):
                slot = blk * L + j
                copies.append(pltpu.async_copy(
                    ufac_hbm.at[tub[j], sub[j]], ut_v.at[slot], sem))
                copies.append(pltpu.async_copy(
                    ifac_hbm.at[tib[j], sib[j]], it_v.at[slot], sem))
        for cp in copies:
            cp.wait()

        for blk in range(C // L):
            acc = zero
            for j in range(L):
                slot = blk * L + j
                p = ut_v[slot, pl.ds(0, L)] * it_v[slot, pl.ds(0, L)]
                for q in range(1, D // L):
                    p += (ut_v[slot, pl.ds(q * L, L)]
                          * it_v[slot, pl.ds(q * L, L)])
                for xi in xor_idx:
                    p = p + _take(p, xi)
                acc = jnp.where(lane == j, p, acc)
            out_v[pl.ds(k * C + blk * L, L)] = acc
        return carry

    lax.fori_loop(0, nchunks, chunk_body, 0)

    pltpu.sync_copy(out_v, out_hbm.at[pl.ds(base, bpw)])


def kernel(user_item_tuple, user_factors, item_factors):
    batch = user_item_tuple.shape[0]
    bpw = batch // NW
    n_tiles = user_factors.shape[0] // 8

    uit = user_item_tuple.astype(jnp.int32)
    u_idx = uit[:, 0]
    i_idx = uit[:, 1]
    ufac3 = user_factors.reshape(n_tiles, 8, D)
    ifac3 = item_factors.reshape(item_factors.shape[0] // 8, 8, D)

    mesh = plsc.VectorSubcoreMesh(core_axis_name="c", subcore_axis_name="s")
    run = functools.partial(
        pl.kernel,
        out_type=jax.ShapeDtypeStruct((batch,), jnp.float32),
        mesh=mesh,
        compiler_params=pltpu.CompilerParams(needs_layout_passes=False),
        scratch_types=[
            pltpu.VMEM((bpw,), jnp.int32),
            pltpu.VMEM((bpw,), jnp.int32),
            pltpu.VMEM((bpw,), jnp.int32),
            pltpu.VMEM((bpw,), jnp.int32),
            pltpu.VMEM((C, D), jnp.float32),
            pltpu.VMEM((C, D), jnp.float32),
            pltpu.VMEM((bpw,), jnp.float32),
            pltpu.SemaphoreType.DMA,
        ],
    )(_dot_body)
    return run(u_idx >> 3, u_idx & 7, i_idx >> 3, i_idx & 7, ufac3, ifac3)
